# Initial kernel scaffold; baseline (speedup 1.0000x reference)
import jax
import jax.numpy as jnp
from jax.experimental import pallas as pl


def kernel(xyz, parent_xyz, feats, skip_feats, W1, b1, W2, b2):
    raise NotImplementedError("write your pallas kernel here")


# speedup vs baseline: 26.2190x; 26.2190x over previous
import functools

import jax
import jax.numpy as jnp
from jax.experimental import pallas as pl
from jax.experimental.pallas import tpu as pltpu

BS, M, N = 8, 1024, 4096
IN_DIM, SKIP_DIM, OUT_DIM = 256, 128, 256
NT = 512


def _body(xyz_ref, pt_ref, f_ref, s_ref, w1a_ref, w1b_ref, b1_ref,
          w2_ref, b2_ref, o_ref):
    xyz = xyz_ref[0]
    pt = pt_ref[0]

    xx = jnp.sum(xyz * xyz, axis=1, keepdims=True)
    sq = pt * pt
    pp = (sq[0:1, :] + sq[1:2, :]) + sq[2:3, :]
    cross = jax.lax.dot_general(xyz, pt, (((1,), (0,)), ((), ())),
                                preferred_element_type=jnp.float32)
    d2 = (pp + xx) - 2.0 * cross

    iota0 = jax.lax.broadcasted_iota(jnp.int32, (M, NT), 0)
    dists = []
    idxs = []
    for _ in range(3):
        dmin = jnp.min(d2, axis=0, keepdims=True)
        sel = d2 == dmin
        idx = jnp.min(jnp.where(sel, iota0, M), axis=0, keepdims=True)
        dists.append(dmin)
        idxs.append(idx)
        d2 = jnp.where(iota0 == idx, jnp.float32(3.4e38), d2)

    inv = [1.0 / (d + 1e-8) for d in dists]
    norm = inv[0] + inv[1] + inv[2]

    S = jnp.zeros((M, NT), jnp.float32)
    for k in range(3):
        S = S + jnp.where(iota0 == idxs[k], inv[k] / norm, 0.0)

    f = f_ref[0]
    f_hi = f.astype(jnp.bfloat16).astype(jnp.float32)
    f_lo = f - f_hi
    s_hi = S.astype(jnp.bfloat16).astype(jnp.float32)
    s_lo = S - s_hi
    interp = (jnp.dot(f_hi, s_hi, preferred_element_type=jnp.float32)
              + jnp.dot(f_lo, s_hi, preferred_element_type=jnp.float32)
              + jnp.dot(f_hi, s_lo, preferred_element_type=jnp.float32))
    h = (jnp.dot(w1a_ref[...], interp, preferred_element_type=jnp.float32)
         + jnp.dot(w1b_ref[...], s_ref[0], preferred_element_type=jnp.float32)
         + b1_ref[...])
    h = jnp.maximum(h, 0.0)
    h = jnp.dot(w2_ref[...], h, preferred_element_type=jnp.float32) + b2_ref[...]
    o_ref[0] = jnp.maximum(h, 0.0)


@jax.jit
def kernel(xyz, parent_xyz, feats, skip_feats, W1, b1, W2, b2):
    parent_t = parent_xyz.transpose(0, 2, 1)
    w1a = W1[:, :IN_DIM]
    w1b = W1[:, IN_DIM:]
    b1c = b1.reshape(OUT_DIM, 1)
    b2c = b2.reshape(OUT_DIM, 1)

    grid = (BS, N // NT)
    out = pl.pallas_call(
        _body,
        grid=grid,
        in_specs=[
            pl.BlockSpec((1, M, 3), lambda b, j: (b, 0, 0)),
            pl.BlockSpec((1, 3, NT), lambda b, j: (b, 0, j)),
            pl.BlockSpec((1, IN_DIM, M), lambda b, j: (b, 0, 0)),
            pl.BlockSpec((1, SKIP_DIM, NT), lambda b, j: (b, 0, j)),
            pl.BlockSpec((OUT_DIM, IN_DIM), lambda b, j: (0, 0)),
            pl.BlockSpec((OUT_DIM, SKIP_DIM), lambda b, j: (0, 0)),
            pl.BlockSpec((OUT_DIM, 1), lambda b, j: (0, 0)),
            pl.BlockSpec((OUT_DIM, OUT_DIM), lambda b, j: (0, 0)),
            pl.BlockSpec((OUT_DIM, 1), lambda b, j: (0, 0)),
        ],
        out_specs=pl.BlockSpec((1, OUT_DIM, NT), lambda b, j: (b, 0, j)),
        out_shape=jax.ShapeDtypeStruct((BS, OUT_DIM, N), jnp.float32),
    )(xyz, parent_t, feats, skip_feats, w1a, w1b, b1c, W2, b2c)
    return out
